# baseline (speedup 1.0000x reference)
import jax
import jax.numpy as jnp
from jax import lax
from jax.experimental import pallas as pl
from jax.experimental.pallas import tpu as pltpu
from jax.experimental.pallas import tpu_sc as plsc

_B = 4
_NGT = 2048
_NSL = (156, 618, 2466)
_NEL = (462, 1848, 7392)
_TS = 512
_NSP = (512, 1024, 2560)
_NEP = (512, 2048, 7680)
_NW = 32
_ECH = tuple(n // _NW for n in _NEP)
_VCH = tuple(n // _NW for n in _NSP)
_BIG = 1e9
_EPS = 1e-12

_NORMAL_W = 0.5
_EDGE_W = 0.1
_LAP_W = 0.5
_MOVE_W = 0.1
_CHAMFER_W = (1.0, 1.0, 1.0)
_CHAMFER_OPP_W = 0.55
_RECONST_W = 0.1
_LAP_CONST = (0.2, 1.0, 1.0)


def _pad_last(x, n_pad, val):
    if x.shape[-1] == n_pad:
        return x
    pad = [(0, 0)] * (x.ndim - 1) + [(0, n_pad - x.shape[-1])]
    return jnp.pad(x, pad, constant_values=val)


_TOT_T = 8
_TOFF = (0, 1, 3)


def _chamfer_body(g_ref, p0_ref, p1_ref, p2_ref, sd1_ref, sd2_ref, idx_ref,
                  dmw_ref):
    t = pl.program_id(1)
    lev = jnp.where(t < 1, 0, jnp.where(t < 3, 1, 2))
    G = g_ref[0]
    pblk = jnp.where(lev == 0, p0_ref[0],
                     jnp.where(lev == 1, p1_ref[0], p2_ref[0]))
    pn = jnp.sum(pblk * pblk, axis=0, keepdims=True)
    P = jnp.concatenate(
        [pblk, jnp.ones((1, _TS), jnp.float32), pn,
         jnp.zeros((3, _TS), jnp.float32)], axis=0)
    d = lax.dot_general(G, P, (((1,), (0,)), ((), ())),
                        precision=lax.Precision.HIGHEST,
                        preferred_element_type=jnp.float32)

    dq = jnp.minimum(jnp.minimum(d[:, 0:128], d[:, 128:256]),
                     jnp.minimum(d[:, 256:384], d[:, 384:512]))
    first = jnp.logical_or(t == 0, jnp.logical_or(t == 1, t == 3))
    last = jnp.logical_or(t == 0, jnp.logical_or(t == 2, t == 7))

    @pl.when(first)
    def _():
        dmw_ref[...] = dq

    @pl.when(jnp.logical_not(first))
    def _():
        dmw_ref[...] = jnp.minimum(dmw_ref[...], dq)

    minv = jnp.min(d, axis=0, keepdims=True)
    iota0 = lax.broadcasted_iota(jnp.int32, d.shape, 0)
    idxm = jnp.min(jnp.where(d == minv, iota0, _NGT), axis=0, keepdims=True)
    idx_ref[0] = idxm

    toff = jnp.where(t < 1, 0, jnp.where(t < 3, 1, 3))
    ns = jnp.where(t < 1, _NSL[0], jnp.where(t < 3, _NSL[1], _NSL[2]))
    lane = lax.broadcasted_iota(jnp.int32, (1, _TS), 1) + (t - toff) * _TS
    s2 = jnp.sum(jnp.where(lane < ns, minv, 0.0))

    @pl.when(first)
    def _():
        sd2_ref[0, 0, lev] = s2

    @pl.when(jnp.logical_not(first))
    def _():
        sd2_ref[0, 0, lev] = sd2_ref[0, 0, lev] + s2

    @pl.when(last)
    def _():
        sd1_ref[0, 0, lev] = jnp.sum(jnp.min(dmw_ref[...], axis=1))


def _chamfer_all(g5, pred_ts):
    npc = _TOT_T * _TS
    return pl.pallas_call(
        _chamfer_body,
        grid=(_B, _TOT_T),
        in_specs=[
            pl.BlockSpec((1, _NGT, 8), lambda b, t: (b, 0, 0)),
            pl.BlockSpec((1, 3, _TS), lambda b, t: (b, 0, 0)),
            pl.BlockSpec((1, 3, _TS),
                         lambda b, t: (b, 0, jnp.clip(t - 1, 0, 1))),
            pl.BlockSpec((1, 3, _TS),
                         lambda b, t: (b, 0, jnp.clip(t - 3, 0, 4))),
        ],
        out_specs=[
            pl.BlockSpec((1, 1, 3), lambda b, t: (b, 0, 0),
                         memory_space=pltpu.SMEM),
            pl.BlockSpec((1, 1, 3), lambda b, t: (b, 0, 0),
                         memory_space=pltpu.SMEM),
            pl.BlockSpec((1, 1, _TS), lambda b, t: (b, 0, t)),
        ],
        out_shape=[
            jax.ShapeDtypeStruct((_B, 1, 3), jnp.float32),
            jax.ShapeDtypeStruct((_B, 1, 3), jnp.float32),
            jax.ShapeDtypeStruct((_B, 1, npc), jnp.int32),
        ],
        scratch_shapes=[pltpu.VMEM((_NGT, 128), jnp.float32)],
    )(g5, pred_ts[0], pred_ts[1], pred_ts[2])


def _sc_body(p0, b0, p1, b1, p2, b2, nrm, i2_0, i2_1, i2_2,
             e00, e10, e01, e11, e02, e12,
             ln0, ln1, ln2, cn0, cn1, cn2,
             tri0, tri1, tri2, parts,
             px, py, pz, bx, by, bz, nx, ny, nz, i2v,
             e0v, e1v, lnv, cntv, av, cv, n2v, accv, sem):
    cid = lax.axis_index("c")
    sid = lax.axis_index("s")
    wid = sid * 2 + cid

    zero16 = jnp.zeros((16,), jnp.float32)
    for q in range(5):
        accv[pl.ds(q * 16, 16)] = zero16

    preds = (p0, p1, p2)
    befs = (b0, b1, b2)
    i2s = (i2_0, i2_1, i2_2)
    e0s = (e00, e01, e02)
    e1s = (e10, e11, e12)
    lns = (ln0, ln1, ln2)
    cns = (cn0, cn1, cn2)
    tris = (tri0, tri1, tri2)

    for b in range(_B):
        hn = [pltpu.async_copy(nrm.at[pl.ds((b * 3 + 0) * _NGT, _NGT)], nx, sem),
              pltpu.async_copy(nrm.at[pl.ds((b * 3 + 1) * _NGT, _NGT)], ny, sem),
              pltpu.async_copy(nrm.at[pl.ds((b * 3 + 2) * _NGT, _NGT)], nz, sem)]
        for h in hn:
            h.wait()
        for lev in range(3):
            nsp = _NSP[lev]
            ech = _ECH[lev]
            vch = _VCH[lev]
            ebase = wid * ech
            vbase = wid * vch
            pb = (b * 3) * nsp
            hs = [
                pltpu.async_copy(preds[lev].at[pl.ds(pb, nsp)],
                                 px.at[pl.ds(0, nsp)], sem),
                pltpu.async_copy(preds[lev].at[pl.ds(pb + nsp, nsp)],
                                 py.at[pl.ds(0, nsp)], sem),
                pltpu.async_copy(preds[lev].at[pl.ds(pb + 2 * nsp, nsp)],
                                 pz.at[pl.ds(0, nsp)], sem),
                pltpu.async_copy(befs[lev].at[pl.ds(pb, nsp)],
                                 bx.at[pl.ds(0, nsp)], sem),
                pltpu.async_copy(befs[lev].at[pl.ds(pb + nsp, nsp)],
                                 by.at[pl.ds(0, nsp)], sem),
                pltpu.async_copy(befs[lev].at[pl.ds(pb + 2 * nsp, nsp)],
                                 bz.at[pl.ds(0, nsp)], sem),
                pltpu.async_copy(i2s[lev].at[pl.ds(b * nsp, nsp)],
                                 i2v.at[pl.ds(0, nsp)], sem),
                pltpu.async_copy(e0s[lev].at[pl.ds(ebase, ech)],
                                 e0v.at[pl.ds(0, ech)], sem),
                pltpu.async_copy(e1s[lev].at[pl.ds(ebase, ech)],
                                 e1v.at[pl.ds(0, ech)], sem),
                pltpu.async_copy(cns[lev].at[pl.ds(vbase, vch)],
                                 cntv.at[pl.ds(0, vch)], sem),
            ]
            for j in range(8):
                hs.append(pltpu.async_copy(
                    lns[lev].at[pl.ds(j * nsp + vbase, vch)],
                    lnv.at[j, pl.ds(0, vch)], sem))
            for h in hs:
                h.wait()

            def edge_iter(i, carry):
                off = i * 16
                e0 = e0v[pl.ds(off, 16)]
                e1 = e1v[pl.ds(off, 16)]
                dx = plsc.load_gather(px, [e0]) - plsc.load_gather(px, [e1])
                dy = plsc.load_gather(py, [e0]) - plsc.load_gather(py, [e1])
                dz = plsc.load_gather(pz, [e0]) - plsc.load_gather(pz, [e1])
                ni = plsc.load_gather(i2v, [e0])
                gx = plsc.load_gather(nx, [ni])
                gy = plsc.load_gather(ny, [ni])
                gz = plsc.load_gather(nz, [ni])
                av[pl.ds(off, 16)] = dx * dx + dy * dy + dz * dz
                cv[pl.ds(off, 16)] = dx * gx + dy * gy + dz * gz
                n2v[pl.ds(off, 16)] = gx * gx + gy * gy + gz * gz
                return carry

            lax.fori_loop(0, ech // 16, edge_iter, 0)
            nep = _NEP[lev]
            tb = (b * 3) * nep + ebase
            ho = [
                pltpu.async_copy(av.at[pl.ds(0, ech)],
                                 tris[lev].at[pl.ds(tb, ech)], sem),
                pltpu.async_copy(cv.at[pl.ds(0, ech)],
                                 tris[lev].at[pl.ds(tb + nep, ech)], sem),
                pltpu.async_copy(n2v.at[pl.ds(0, ech)],
                                 tris[lev].at[pl.ds(tb + 2 * nep, ech)], sem),
            ]

            _LAPQ = lev
            _MOVQ = 2 + lev if lev > 0 else -1

            def lap_iter(i, carry):
                off = i * 16
                g = vbase + off
                dxv = bx[pl.ds(g, 16)] - px[pl.ds(g, 16)]
                dyv = by[pl.ds(g, 16)] - py[pl.ds(g, 16)]
                dzv = bz[pl.ds(g, 16)] - pz[pl.ds(g, 16)]
                sx = jnp.zeros((16,), jnp.float32)
                sy = jnp.zeros((16,), jnp.float32)
                sz = jnp.zeros((16,), jnp.float32)
                for j in range(8):
                    nb = lnv[j, pl.ds(off, 16)]
                    vf = jnp.where(nb >= 0, 1.0, 0.0).astype(jnp.float32)
                    nbs = jnp.maximum(nb, 0)
                    sx = sx + (plsc.load_gather(bx, [nbs])
                               - plsc.load_gather(px, [nbs])) * vf
                    sy = sy + (plsc.load_gather(by, [nbs])
                               - plsc.load_gather(py, [nbs])) * vf
                    sz = sz + (plsc.load_gather(bz, [nbs])
                               - plsc.load_gather(pz, [nbs])) * vf
                cc = cntv[pl.ds(off, 16)]
                lx = dxv - sx / cc
                ly = dyv - sy / cc
                lz = dzv - sz / cc
                lo = _LAPQ * 16
                accv[pl.ds(lo, 16)] = (accv[pl.ds(lo, 16)]
                                       + lx * lx + ly * ly + lz * lz)
                if _MOVQ >= 0:
                    mo = _MOVQ * 16
                    accv[pl.ds(mo, 16)] = (accv[pl.ds(mo, 16)]
                                           + dxv * dxv + dyv * dyv + dzv * dzv)
                return carry

            lax.fori_loop(0, vch // 16, lap_iter, 0)
            for h in ho:
                h.wait()

    pltpu.sync_copy(accv, parts.at[pl.ds(wid * 80, 80)])


def _sc_gather(pred_ts, bef_ts, nrm_t, idx2s, e0s, e1s, lapns, cnts):
    mesh = plsc.VectorSubcoreMesh(core_axis_name="c", subcore_axis_name="s",
                                  num_cores=2, num_subcores=16)
    out_type = [
        jax.ShapeDtypeStruct((_B * 3 * _NEP[0],), jnp.float32),
        jax.ShapeDtypeStruct((_B * 3 * _NEP[1],), jnp.float32),
        jax.ShapeDtypeStruct((_B * 3 * _NEP[2],), jnp.float32),
        jax.ShapeDtypeStruct((_NW * 80,), jnp.float32),
    ]
    scratch = [
        pltpu.VMEM((2560,), jnp.float32),
        pltpu.VMEM((2560,), jnp.float32),
        pltpu.VMEM((2560,), jnp.float32),
        pltpu.VMEM((2560,), jnp.float32),
        pltpu.VMEM((2560,), jnp.float32),
        pltpu.VMEM((2560,), jnp.float32),
        pltpu.VMEM((2048,), jnp.float32),
        pltpu.VMEM((2048,), jnp.float32),
        pltpu.VMEM((2048,), jnp.float32),
        pltpu.VMEM((2560,), jnp.int32),
        pltpu.VMEM((256,), jnp.int32),
        pltpu.VMEM((256,), jnp.int32),
        pltpu.VMEM((8, 128), jnp.int32),
        pltpu.VMEM((128,), jnp.float32),
        pltpu.VMEM((256,), jnp.float32),
        pltpu.VMEM((256,), jnp.float32),
        pltpu.VMEM((256,), jnp.float32),
        pltpu.VMEM((80,), jnp.float32),
        pltpu.SemaphoreType.DMA,
    ]
    f = pl.kernel(_sc_body, out_type=out_type, mesh=mesh,
                  scratch_types=scratch,
                  compiler_params=pltpu.CompilerParams(
                      use_tc_tiling_on_sc=False,
                      needs_layout_passes=False))
    return f(pred_ts[0], bef_ts[0], pred_ts[1], bef_ts[1],
             pred_ts[2], bef_ts[2], nrm_t,
             idx2s[0], idx2s[1], idx2s[2],
             e0s[0], e1s[0], e0s[1], e1s[1], e0s[2], e1s[2],
             lapns[0], lapns[1], lapns[2], cnts[0], cnts[1], cnts[2])


def _bce_body(gt_ref, p_ref, out_ref):
    p = jnp.clip(p_ref[...], 1e-7, 1.0 - 1e-7)
    gt = gt_ref[...]
    out_ref[0, 0, 0] = jnp.sum(gt * jnp.log(p)
                               + (1.0 - gt) * jnp.log(1.0 - p))


def _bce(gt_img, pred_img):
    return pl.pallas_call(
        _bce_body,
        grid=(3,),
        in_specs=[
            pl.BlockSpec((392, 512), lambda i: (i, 0)),
            pl.BlockSpec((392, 512), lambda i: (i, 0)),
        ],
        out_specs=pl.BlockSpec((1, 1, 1), lambda i: (i, 0, 0),
                               memory_space=pltpu.SMEM),
        out_shape=jax.ShapeDtypeStruct((3, 1, 1), jnp.float32),
        compiler_params=pltpu.CompilerParams(
            dimension_semantics=("parallel",)),
    )(gt_img, pred_img)


def _combine_body(sd1a, sd2a, tri0, tri1, tri2, parts, bs, *outs):
    sd1 = tuple(sum(sd1a[b, 0, i] for b in range(_B)) for i in range(3))
    sd2 = tuple(sum(sd2a[b, 0, i] for b in range(_B)) for i in range(3))
    tris = (tri0, tri1, tri2)
    chamfer = jnp.float32(0.0)
    edge = jnp.float32(0.0)
    normal = jnp.float32(0.0)
    lap = jnp.float32(0.0)
    move = jnp.float32(0.0)
    for i in range(3):
        ns = _NSL[i]
        ne = _NEL[i]
        a = tris[i][:, 0, :]
        c = tris[i][:, 1, :]
        n2 = tris[i][:, 2, :]
        cos = jnp.abs(c) / (jnp.maximum(jnp.sqrt(a), _EPS)
                            * jnp.maximum(jnp.sqrt(n2), _EPS))
        normal = normal + jnp.sum(cos) / (_B * ne)
        edge = edge + jnp.sum(a) / (_B * ne)
        chamfer = chamfer + _CHAMFER_W[i] * (
            sd1[i] / (_B * _NGT) + _CHAMFER_OPP_W * sd2[i] / (_B * ns))
        lap = lap + _LAP_CONST[i] * jnp.sum(parts[:, i, :]) / (_B * ns)
        if i > 0:
            move = move + _LAP_CONST[i] * jnp.sum(parts[:, 2 + i, :]) / (_B * ns)
    image = -(bs[0, 0, 0] + bs[1, 0, 0] + bs[2, 0, 0]) / (_B * 3 * 224 * 224)
    loss = (chamfer + image * _RECONST_W + _LAP_W * lap + _MOVE_W * move
            + _EDGE_W * edge + _NORMAL_W * normal)
    vals = (loss, image, chamfer, edge, lap, move, normal)
    for r, v in zip(outs, vals):
        r[0, 0] = v


def _combine(sd1a, sd2a, tris, parts, bs):
    smem = pl.BlockSpec(memory_space=pltpu.SMEM)
    in_specs = [smem, smem] + [pl.BlockSpec(t.shape, lambda: (0, 0, 0))
                               for t in tris]
    in_specs += [pl.BlockSpec(parts.shape, lambda: (0, 0, 0)), smem]
    return pl.pallas_call(
        _combine_body,
        in_specs=in_specs,
        out_specs=[smem] * 7,
        out_shape=[jax.ShapeDtypeStruct((1, 1), jnp.float32)] * 7,
    )(sd1a, sd2a, *tris, parts, bs)


def kernel(gt_points, gt_normals, gt_images, pred_reconst,
           pred_coord_0, pred_coord_1, pred_coord_2,
           pred_before_0, pred_before_1, pred_before_2,
           edges_0, edges_1, edges_2,
           lap_idx_0, lap_idx_1, lap_idx_2):
    gt_p = gt_points.astype(jnp.float32)
    nrm_t = jnp.transpose(gt_normals, (0, 2, 1)).astype(jnp.float32)
    preds = (pred_coord_0, pred_coord_1, pred_coord_2)
    befs = (pred_before_0, pred_before_1, pred_before_2)
    edges = (edges_0, edges_1, edges_2)
    laps = (lap_idx_0, lap_idx_1, lap_idx_2)

    pred_ts, bef_ts, e0s, e1s, lapns, cnts = [], [], [], [], [], []
    for i in range(3):
        nsp = _NSP[i]
        nep = _NEP[i]
        pred_t = _pad_last(jnp.transpose(preds[i], (0, 2, 1)), nsp, _BIG)
        bef_t = _pad_last(jnp.transpose(befs[i], (0, 2, 1)), nsp, _BIG)
        e = edges[i].astype(jnp.int32)
        e0 = _pad_last(e[:, 0], nep, 0)
        e1 = _pad_last(e[:, 1], nep, 0)
        li = laps[i].astype(jnp.int32)
        lapn = _pad_last(jnp.transpose(li[:, :8], (1, 0)), nsp, -1)
        cnt = _pad_last(li[:, 9].astype(jnp.float32), nsp, 1.0)
        pred_ts.append(pred_t)
        bef_ts.append(bef_t)
        e0s.append(e0)
        e1s.append(e1)
        lapns.append(lapn)
        cnts.append(cnt)

    gn = jnp.sum(gt_p * gt_p, axis=-1, keepdims=True)
    g5 = jnp.concatenate(
        [-2.0 * gt_p, gn, jnp.ones((_B, _NGT, 1), jnp.float32),
         jnp.zeros((_B, _NGT, 3), jnp.float32)], axis=-1)
    sd1a, sd2a, idx2c = _chamfer_all(g5, pred_ts)
    idx2s = [idx2c[:, :, _TOFF[i] * _TS:_TOFF[i] * _TS + _NSP[i]]
             for i in range(3)]

    pred_fs = [jnp.reshape(p, (-1,)) for p in pred_ts]
    bef_fs = [jnp.reshape(p, (-1,)) for p in bef_ts]
    nrm_f = jnp.reshape(nrm_t, (-1,))
    idx2_fs = [jnp.reshape(ix, (-1,)) for ix in idx2s]
    lapn_fs = [jnp.reshape(ln, (-1,)) for ln in lapns]
    tri0, tri1, tri2, parts = _sc_gather(pred_fs, bef_fs, nrm_f, idx2_fs,
                                         e0s, e1s, lapn_fs, cnts)
    tri0 = jnp.reshape(tri0, (_B, 3, _NEP[0]))
    tri1 = jnp.reshape(tri1, (_B, 3, _NEP[1]))
    tri2 = jnp.reshape(tri2, (_B, 3, _NEP[2]))
    parts = jnp.reshape(parts, (_NW, 5, 16))

    gt_img = jnp.reshape(gt_images, (1176, 512))
    p_img = jnp.reshape(pred_reconst, (1176, 512))
    bs = _bce(gt_img, p_img)

    outs = _combine(sd1a, sd2a, (tri0, tri1, tri2), parts, bs)
    return tuple(jnp.reshape(o, ()) for o in outs)


# speedup vs baseline: 1.2459x; 1.2459x over previous
import jax
import jax.numpy as jnp
from jax import lax
from jax.experimental import pallas as pl
from jax.experimental.pallas import tpu as pltpu
from jax.experimental.pallas import tpu_sc as plsc

_B = 4
_NGT = 2048
_NSL = (156, 618, 2466)
_NEL = (462, 1848, 7392)
_TS = 512
_NSP = (512, 1024, 2560)
_NEP = (512, 2048, 7680)
_NW = 32
_ECH = tuple(n // _NW for n in _NEP)
_VCH = tuple(n // _NW for n in _NSP)
_BIG = 1e9
_EPS = 1e-12

_NORMAL_W = 0.5
_EDGE_W = 0.1
_LAP_W = 0.5
_MOVE_W = 0.1
_CHAMFER_W = (1.0, 1.0, 1.0)
_CHAMFER_OPP_W = 0.55
_RECONST_W = 0.1
_LAP_CONST = (0.2, 1.0, 1.0)


def _pad_last(x, n_pad, val):
    if x.shape[-1] == n_pad:
        return x
    pad = [(0, 0)] * (x.ndim - 1) + [(0, n_pad - x.shape[-1])]
    return jnp.pad(x, pad, constant_values=val)


_TOT_T = 8
_TOFF = (0, 1, 3)


def _chamfer_body(g1_ref, g2_ref, p0_ref, p1_ref, p2_ref, sd1_ref, sd2_ref,
                  idx_ref, dmw_ref):
    t = pl.program_id(1)
    lev = jnp.where(t < 1, 0, jnp.where(t < 3, 1, 2))
    g1 = g1_ref[0]
    g2 = g2_ref[0]
    pblk = jnp.where(lev == 0, p0_ref[0],
                     jnp.where(lev == 1, p1_ref[0], p2_ref[0]))
    pn = jnp.sum(pblk * pblk, axis=0, keepdims=True)
    P = jnp.concatenate(
        [pblk, jnp.ones((1, _TS), jnp.float32), pn,
         jnp.zeros((3, _TS), jnp.float32)], axis=0)
    dn = (((1,), (0,)), ((), ()))
    b1 = P.astype(jnp.bfloat16)
    b2 = (P - b1.astype(jnp.float32)).astype(jnp.bfloat16)
    d = (lax.dot_general(g1, b1, dn, preferred_element_type=jnp.float32)
         + (lax.dot_general(g1, b2, dn, preferred_element_type=jnp.float32)
            + lax.dot_general(g2, b1, dn,
                              preferred_element_type=jnp.float32)))

    dq = jnp.minimum(jnp.minimum(d[:, 0:128], d[:, 128:256]),
                     jnp.minimum(d[:, 256:384], d[:, 384:512]))
    first = jnp.logical_or(t == 0, jnp.logical_or(t == 1, t == 3))
    last = jnp.logical_or(t == 0, jnp.logical_or(t == 2, t == 7))

    @pl.when(first)
    def _():
        dmw_ref[...] = dq

    @pl.when(jnp.logical_not(first))
    def _():
        dmw_ref[...] = jnp.minimum(dmw_ref[...], dq)

    minv = jnp.min(d, axis=0, keepdims=True)
    iota0 = lax.broadcasted_iota(jnp.int32, d.shape, 0)
    idxm = jnp.min(jnp.where(d == minv, iota0, _NGT), axis=0, keepdims=True)
    idx_ref[0] = idxm

    toff = jnp.where(t < 1, 0, jnp.where(t < 3, 1, 3))
    ns = jnp.where(t < 1, _NSL[0], jnp.where(t < 3, _NSL[1], _NSL[2]))
    lane = lax.broadcasted_iota(jnp.int32, (1, _TS), 1) + (t - toff) * _TS
    s2 = jnp.sum(jnp.where(lane < ns, minv, 0.0))

    @pl.when(first)
    def _():
        sd2_ref[0, 0, lev] = s2

    @pl.when(jnp.logical_not(first))
    def _():
        sd2_ref[0, 0, lev] = sd2_ref[0, 0, lev] + s2

    @pl.when(last)
    def _():
        sd1_ref[0, 0, lev] = jnp.sum(jnp.min(dmw_ref[...], axis=1))


def _chamfer_all(g5hi, g5lo, pred_ts):
    npc = _TOT_T * _TS
    return pl.pallas_call(
        _chamfer_body,
        grid=(_B, _TOT_T),
        in_specs=[
            pl.BlockSpec((1, _NGT, 8), lambda b, t: (b, 0, 0)),
            pl.BlockSpec((1, _NGT, 8), lambda b, t: (b, 0, 0)),
            pl.BlockSpec((1, 3, _TS), lambda b, t: (b, 0, 0)),
            pl.BlockSpec((1, 3, _TS),
                         lambda b, t: (b, 0, jnp.clip(t - 1, 0, 1))),
            pl.BlockSpec((1, 3, _TS),
                         lambda b, t: (b, 0, jnp.clip(t - 3, 0, 4))),
        ],
        out_specs=[
            pl.BlockSpec((1, 1, 3), lambda b, t: (b, 0, 0),
                         memory_space=pltpu.SMEM),
            pl.BlockSpec((1, 1, 3), lambda b, t: (b, 0, 0),
                         memory_space=pltpu.SMEM),
            pl.BlockSpec((1, 1, _TS), lambda b, t: (b, 0, t)),
        ],
        out_shape=[
            jax.ShapeDtypeStruct((_B, 1, 3), jnp.float32),
            jax.ShapeDtypeStruct((_B, 1, 3), jnp.float32),
            jax.ShapeDtypeStruct((_B, 1, npc), jnp.int32),
        ],
        scratch_shapes=[pltpu.VMEM((_NGT, 128), jnp.float32)],
    )(g5hi, g5lo, pred_ts[0], pred_ts[1], pred_ts[2])


def _sc_body(p0, b0, p1, b1, p2, b2, nrm, i2_0, i2_1, i2_2,
             e00, e10, e01, e11, e02, e12,
             ln0, ln1, ln2, cn0, cn1, cn2,
             tri0, tri1, tri2, parts,
             px, py, pz, bx, by, bz, nx, ny, nz, i2v,
             e0v, e1v, lnv, cntv, av, cv, n2v, accv, sem):
    cid = lax.axis_index("c")
    sid = lax.axis_index("s")
    wid = sid * 2 + cid

    zero16 = jnp.zeros((16,), jnp.float32)
    for q in range(5):
        accv[pl.ds(q * 16, 16)] = zero16

    preds = (p0, p1, p2)
    befs = (b0, b1, b2)
    i2s = (i2_0, i2_1, i2_2)
    e0s = (e00, e01, e02)
    e1s = (e10, e11, e12)
    lns = (ln0, ln1, ln2)
    cns = (cn0, cn1, cn2)
    tris = (tri0, tri1, tri2)

    for b in range(_B):
        hn = [pltpu.async_copy(nrm.at[pl.ds((b * 3 + 0) * _NGT, _NGT)], nx, sem),
              pltpu.async_copy(nrm.at[pl.ds((b * 3 + 1) * _NGT, _NGT)], ny, sem),
              pltpu.async_copy(nrm.at[pl.ds((b * 3 + 2) * _NGT, _NGT)], nz, sem)]
        for h in hn:
            h.wait()
        for lev in range(3):
            nsp = _NSP[lev]
            ech = _ECH[lev]
            vch = _VCH[lev]
            ebase = wid * ech
            vbase = wid * vch
            pb = (b * 3) * nsp
            hs = [
                pltpu.async_copy(preds[lev].at[pl.ds(pb, nsp)],
                                 px.at[pl.ds(0, nsp)], sem),
                pltpu.async_copy(preds[lev].at[pl.ds(pb + nsp, nsp)],
                                 py.at[pl.ds(0, nsp)], sem),
                pltpu.async_copy(preds[lev].at[pl.ds(pb + 2 * nsp, nsp)],
                                 pz.at[pl.ds(0, nsp)], sem),
                pltpu.async_copy(befs[lev].at[pl.ds(pb, nsp)],
                                 bx.at[pl.ds(0, nsp)], sem),
                pltpu.async_copy(befs[lev].at[pl.ds(pb + nsp, nsp)],
                                 by.at[pl.ds(0, nsp)], sem),
                pltpu.async_copy(befs[lev].at[pl.ds(pb + 2 * nsp, nsp)],
                                 bz.at[pl.ds(0, nsp)], sem),
                pltpu.async_copy(i2s[lev].at[pl.ds(b * nsp, nsp)],
                                 i2v.at[pl.ds(0, nsp)], sem),
                pltpu.async_copy(e0s[lev].at[pl.ds(ebase, ech)],
                                 e0v.at[pl.ds(0, ech)], sem),
                pltpu.async_copy(e1s[lev].at[pl.ds(ebase, ech)],
                                 e1v.at[pl.ds(0, ech)], sem),
                pltpu.async_copy(cns[lev].at[pl.ds(vbase, vch)],
                                 cntv.at[pl.ds(0, vch)], sem),
            ]
            for j in range(8):
                hs.append(pltpu.async_copy(
                    lns[lev].at[pl.ds(j * nsp + vbase, vch)],
                    lnv.at[j, pl.ds(0, vch)], sem))
            for h in hs:
                h.wait()

            def edge_iter(i, carry):
                off = i * 16
                e0 = e0v[pl.ds(off, 16)]
                e1 = e1v[pl.ds(off, 16)]
                dx = plsc.load_gather(px, [e0]) - plsc.load_gather(px, [e1])
                dy = plsc.load_gather(py, [e0]) - plsc.load_gather(py, [e1])
                dz = plsc.load_gather(pz, [e0]) - plsc.load_gather(pz, [e1])
                ni = plsc.load_gather(i2v, [e0])
                gx = plsc.load_gather(nx, [ni])
                gy = plsc.load_gather(ny, [ni])
                gz = plsc.load_gather(nz, [ni])
                av[pl.ds(off, 16)] = dx * dx + dy * dy + dz * dz
                cv[pl.ds(off, 16)] = dx * gx + dy * gy + dz * gz
                n2v[pl.ds(off, 16)] = gx * gx + gy * gy + gz * gz
                return carry

            lax.fori_loop(0, ech // 16, edge_iter, 0)
            nep = _NEP[lev]
            tb = (b * 3) * nep + ebase
            ho = [
                pltpu.async_copy(av.at[pl.ds(0, ech)],
                                 tris[lev].at[pl.ds(tb, ech)], sem),
                pltpu.async_copy(cv.at[pl.ds(0, ech)],
                                 tris[lev].at[pl.ds(tb + nep, ech)], sem),
                pltpu.async_copy(n2v.at[pl.ds(0, ech)],
                                 tris[lev].at[pl.ds(tb + 2 * nep, ech)], sem),
            ]

            _LAPQ = lev
            _MOVQ = 2 + lev if lev > 0 else -1

            def lap_iter(i, carry):
                off = i * 16
                g = vbase + off
                dxv = bx[pl.ds(g, 16)] - px[pl.ds(g, 16)]
                dyv = by[pl.ds(g, 16)] - py[pl.ds(g, 16)]
                dzv = bz[pl.ds(g, 16)] - pz[pl.ds(g, 16)]
                sx = jnp.zeros((16,), jnp.float32)
                sy = jnp.zeros((16,), jnp.float32)
                sz = jnp.zeros((16,), jnp.float32)
                for j in range(8):
                    nb = lnv[j, pl.ds(off, 16)]
                    vf = jnp.where(nb >= 0, 1.0, 0.0).astype(jnp.float32)
                    nbs = jnp.maximum(nb, 0)
                    sx = sx + (plsc.load_gather(bx, [nbs])
                               - plsc.load_gather(px, [nbs])) * vf
                    sy = sy + (plsc.load_gather(by, [nbs])
                               - plsc.load_gather(py, [nbs])) * vf
                    sz = sz + (plsc.load_gather(bz, [nbs])
                               - plsc.load_gather(pz, [nbs])) * vf
                cc = cntv[pl.ds(off, 16)]
                lx = dxv - sx / cc
                ly = dyv - sy / cc
                lz = dzv - sz / cc
                lo = _LAPQ * 16
                accv[pl.ds(lo, 16)] = (accv[pl.ds(lo, 16)]
                                       + lx * lx + ly * ly + lz * lz)
                if _MOVQ >= 0:
                    mo = _MOVQ * 16
                    accv[pl.ds(mo, 16)] = (accv[pl.ds(mo, 16)]
                                           + dxv * dxv + dyv * dyv + dzv * dzv)
                return carry

            lax.fori_loop(0, vch // 16, lap_iter, 0)
            for h in ho:
                h.wait()

    pltpu.sync_copy(accv, parts.at[pl.ds(wid * 80, 80)])


def _sc_gather(pred_ts, bef_ts, nrm_t, idx2s, e0s, e1s, lapns, cnts):
    mesh = plsc.VectorSubcoreMesh(core_axis_name="c", subcore_axis_name="s",
                                  num_cores=2, num_subcores=16)
    out_type = [
        jax.ShapeDtypeStruct((_B * 3 * _NEP[0],), jnp.float32),
        jax.ShapeDtypeStruct((_B * 3 * _NEP[1],), jnp.float32),
        jax.ShapeDtypeStruct((_B * 3 * _NEP[2],), jnp.float32),
        jax.ShapeDtypeStruct((_NW * 80,), jnp.float32),
    ]
    scratch = [
        pltpu.VMEM((2560,), jnp.float32),
        pltpu.VMEM((2560,), jnp.float32),
        pltpu.VMEM((2560,), jnp.float32),
        pltpu.VMEM((2560,), jnp.float32),
        pltpu.VMEM((2560,), jnp.float32),
        pltpu.VMEM((2560,), jnp.float32),
        pltpu.VMEM((2048,), jnp.float32),
        pltpu.VMEM((2048,), jnp.float32),
        pltpu.VMEM((2048,), jnp.float32),
        pltpu.VMEM((2560,), jnp.int32),
        pltpu.VMEM((256,), jnp.int32),
        pltpu.VMEM((256,), jnp.int32),
        pltpu.VMEM((8, 128), jnp.int32),
        pltpu.VMEM((128,), jnp.float32),
        pltpu.VMEM((256,), jnp.float32),
        pltpu.VMEM((256,), jnp.float32),
        pltpu.VMEM((256,), jnp.float32),
        pltpu.VMEM((80,), jnp.float32),
        pltpu.SemaphoreType.DMA,
    ]
    f = pl.kernel(_sc_body, out_type=out_type, mesh=mesh,
                  scratch_types=scratch,
                  compiler_params=pltpu.CompilerParams(
                      use_tc_tiling_on_sc=False,
                      needs_layout_passes=False))
    return f(pred_ts[0], bef_ts[0], pred_ts[1], bef_ts[1],
             pred_ts[2], bef_ts[2], nrm_t,
             idx2s[0], idx2s[1], idx2s[2],
             e0s[0], e1s[0], e0s[1], e1s[1], e0s[2], e1s[2],
             lapns[0], lapns[1], lapns[2], cnts[0], cnts[1], cnts[2])


def _bce_body(gt_ref, p_ref, out_ref):
    p = jnp.clip(p_ref[...], 1e-7, 1.0 - 1e-7)
    gt = gt_ref[...]
    out_ref[0, 0, 0] = jnp.sum(gt * jnp.log(p)
                               + (1.0 - gt) * jnp.log(1.0 - p))


def _bce(gt_img, pred_img):
    return pl.pallas_call(
        _bce_body,
        grid=(3,),
        in_specs=[
            pl.BlockSpec((392, 512), lambda i: (i, 0)),
            pl.BlockSpec((392, 512), lambda i: (i, 0)),
        ],
        out_specs=pl.BlockSpec((1, 1, 1), lambda i: (i, 0, 0),
                               memory_space=pltpu.SMEM),
        out_shape=jax.ShapeDtypeStruct((3, 1, 1), jnp.float32),
        compiler_params=pltpu.CompilerParams(
            dimension_semantics=("parallel",)),
    )(gt_img, pred_img)


def _combine_body(sd1a, sd2a, tri0, tri1, tri2, parts, bs, *outs):
    sd1 = tuple(sum(sd1a[b, 0, i] for b in range(_B)) for i in range(3))
    sd2 = tuple(sum(sd2a[b, 0, i] for b in range(_B)) for i in range(3))
    tris = (tri0, tri1, tri2)
    chamfer = jnp.float32(0.0)
    edge = jnp.float32(0.0)
    normal = jnp.float32(0.0)
    lap = jnp.float32(0.0)
    move = jnp.float32(0.0)
    for i in range(3):
        ns = _NSL[i]
        ne = _NEL[i]
        a = tris[i][:, 0, :]
        c = tris[i][:, 1, :]
        n2 = tris[i][:, 2, :]
        cos = jnp.abs(c) / (jnp.maximum(jnp.sqrt(a), _EPS)
                            * jnp.maximum(jnp.sqrt(n2), _EPS))
        normal = normal + jnp.sum(cos) / (_B * ne)
        edge = edge + jnp.sum(a) / (_B * ne)
        chamfer = chamfer + _CHAMFER_W[i] * (
            sd1[i] / (_B * _NGT) + _CHAMFER_OPP_W * sd2[i] / (_B * ns))
        lap = lap + _LAP_CONST[i] * jnp.sum(parts[:, i, :]) / (_B * ns)
        if i > 0:
            move = move + _LAP_CONST[i] * jnp.sum(parts[:, 2 + i, :]) / (_B * ns)
    image = -(bs[0, 0, 0] + bs[1, 0, 0] + bs[2, 0, 0]) / (_B * 3 * 224 * 224)
    loss = (chamfer + image * _RECONST_W + _LAP_W * lap + _MOVE_W * move
            + _EDGE_W * edge + _NORMAL_W * normal)
    vals = (loss, image, chamfer, edge, lap, move, normal)
    for r, v in zip(outs, vals):
        r[0, 0] = v


def _combine(sd1a, sd2a, tris, parts, bs):
    smem = pl.BlockSpec(memory_space=pltpu.SMEM)
    in_specs = [smem, smem] + [pl.BlockSpec(t.shape, lambda: (0, 0, 0))
                               for t in tris]
    in_specs += [pl.BlockSpec(parts.shape, lambda: (0, 0, 0)), smem]
    return pl.pallas_call(
        _combine_body,
        in_specs=in_specs,
        out_specs=[smem] * 7,
        out_shape=[jax.ShapeDtypeStruct((1, 1), jnp.float32)] * 7,
    )(sd1a, sd2a, *tris, parts, bs)


def kernel(gt_points, gt_normals, gt_images, pred_reconst,
           pred_coord_0, pred_coord_1, pred_coord_2,
           pred_before_0, pred_before_1, pred_before_2,
           edges_0, edges_1, edges_2,
           lap_idx_0, lap_idx_1, lap_idx_2):
    gt_p = gt_points.astype(jnp.float32)
    nrm_t = jnp.transpose(gt_normals, (0, 2, 1)).astype(jnp.float32)
    preds = (pred_coord_0, pred_coord_1, pred_coord_2)
    befs = (pred_before_0, pred_before_1, pred_before_2)
    edges = (edges_0, edges_1, edges_2)
    laps = (lap_idx_0, lap_idx_1, lap_idx_2)

    pred_ts, bef_ts, e0s, e1s, lapns, cnts = [], [], [], [], [], []
    for i in range(3):
        nsp = _NSP[i]
        nep = _NEP[i]
        pred_t = _pad_last(jnp.transpose(preds[i], (0, 2, 1)), nsp, _BIG)
        bef_t = _pad_last(jnp.transpose(befs[i], (0, 2, 1)), nsp, _BIG)
        e = edges[i].astype(jnp.int32)
        e0 = _pad_last(e[:, 0], nep, 0)
        e1 = _pad_last(e[:, 1], nep, 0)
        li = laps[i].astype(jnp.int32)
        lapn = _pad_last(jnp.transpose(li[:, :8], (1, 0)), nsp, -1)
        cnt = _pad_last(li[:, 9].astype(jnp.float32), nsp, 1.0)
        pred_ts.append(pred_t)
        bef_ts.append(bef_t)
        e0s.append(e0)
        e1s.append(e1)
        lapns.append(lapn)
        cnts.append(cnt)

    gn = jnp.sum(gt_p * gt_p, axis=-1, keepdims=True)
    g5 = jnp.concatenate(
        [-2.0 * gt_p, gn, jnp.ones((_B, _NGT, 1), jnp.float32),
         jnp.zeros((_B, _NGT, 3), jnp.float32)], axis=-1)
    g5hi = g5.astype(jnp.bfloat16)
    g5lo = (g5 - g5hi.astype(jnp.float32)).astype(jnp.bfloat16)
    sd1a, sd2a, idx2c = _chamfer_all(g5hi, g5lo, pred_ts)
    idx2s = [idx2c[:, :, _TOFF[i] * _TS:_TOFF[i] * _TS + _NSP[i]]
             for i in range(3)]

    pred_fs = [jnp.reshape(p, (-1,)) for p in pred_ts]
    bef_fs = [jnp.reshape(p, (-1,)) for p in bef_ts]
    nrm_f = jnp.reshape(nrm_t, (-1,))
    idx2_fs = [jnp.reshape(ix, (-1,)) for ix in idx2s]
    lapn_fs = [jnp.reshape(ln, (-1,)) for ln in lapns]
    tri0, tri1, tri2, parts = _sc_gather(pred_fs, bef_fs, nrm_f, idx2_fs,
                                         e0s, e1s, lapn_fs, cnts)
    tri0 = jnp.reshape(tri0, (_B, 3, _NEP[0]))
    tri1 = jnp.reshape(tri1, (_B, 3, _NEP[1]))
    tri2 = jnp.reshape(tri2, (_B, 3, _NEP[2]))
    parts = jnp.reshape(parts, (_NW, 5, 16))

    gt_img = jnp.reshape(gt_images, (1176, 512))
    p_img = jnp.reshape(pred_reconst, (1176, 512))
    bs = _bce(gt_img, p_img)

    outs = _combine(sd1a, sd2a, (tri0, tri1, tri2), parts, bs)
    return tuple(jnp.reshape(o, ()) for o in outs)


# speedup vs baseline: 1.6090x; 1.2914x over previous
import jax
import jax.numpy as jnp
from jax import lax
from jax.experimental import pallas as pl
from jax.experimental.pallas import tpu as pltpu
from jax.experimental.pallas import tpu_sc as plsc

_B = 4
_NGT = 2048
_NSL = (156, 618, 2466)
_NEL = (462, 1848, 7392)
_TS = 512
_NSP = (512, 1024, 2560)
_NEP = (512, 2048, 7680)
_NW = 32
_ECH = tuple(n // _NW for n in _NEP)
_VCH = tuple(n // _NW for n in _NSP)
_BIG = 1e9
_EPS = 1e-12

_NORMAL_W = 0.5
_EDGE_W = 0.1
_LAP_W = 0.5
_MOVE_W = 0.1
_CHAMFER_W = (1.0, 1.0, 1.0)
_CHAMFER_OPP_W = 0.55
_RECONST_W = 0.1
_LAP_CONST = (0.2, 1.0, 1.0)


def _pad_last(x, n_pad, val):
    if x.shape[-1] == n_pad:
        return x
    pad = [(0, 0)] * (x.ndim - 1) + [(0, n_pad - x.shape[-1])]
    return jnp.pad(x, pad, constant_values=val)


_TOT_T = 8
_TOFF = (0, 1, 3)


def _chamfer_body(g_ref, p0_ref, p1_ref, p2_ref, sd1_ref, sd2_ref,
                  idx_ref, dmw_ref):
    t = pl.program_id(1)
    lev = jnp.where(t < 1, 0, jnp.where(t < 3, 1, 2))
    g24 = g_ref[0]
    pblk = jnp.where(lev == 0, p0_ref[0],
                     jnp.where(lev == 1, p1_ref[0], p2_ref[0]))
    pn = jnp.sum(pblk * pblk, axis=0, keepdims=True)
    P = jnp.concatenate(
        [pblk, jnp.ones((1, _TS), jnp.float32), pn,
         jnp.zeros((3, _TS), jnp.float32)], axis=0)
    dn = (((1,), (0,)), ((), ()))
    b1 = P.astype(jnp.bfloat16)
    b2 = (P - b1.astype(jnp.float32)).astype(jnp.bfloat16)
    b24 = jnp.concatenate([b1, b2, b1], axis=0)
    d = lax.dot_general(g24, b24, dn,
                        preferred_element_type=jnp.float32)

    dq = jnp.minimum(jnp.minimum(d[:, 0:128], d[:, 128:256]),
                     jnp.minimum(d[:, 256:384], d[:, 384:512]))
    first = jnp.logical_or(t == 0, jnp.logical_or(t == 1, t == 3))
    last = jnp.logical_or(t == 0, jnp.logical_or(t == 2, t == 7))

    @pl.when(first)
    def _():
        dmw_ref[...] = dq

    @pl.when(jnp.logical_not(first))
    def _():
        dmw_ref[...] = jnp.minimum(dmw_ref[...], dq)

    minv = jnp.min(d, axis=0, keepdims=True)
    iota0 = lax.broadcasted_iota(jnp.int32, d.shape, 0)
    idxm = jnp.min(jnp.where(d == minv, iota0, _NGT), axis=0, keepdims=True)
    idx_ref[0] = idxm

    toff = jnp.where(t < 1, 0, jnp.where(t < 3, 1, 3))
    ns = jnp.where(t < 1, _NSL[0], jnp.where(t < 3, _NSL[1], _NSL[2]))
    lane = lax.broadcasted_iota(jnp.int32, (1, _TS), 1) + (t - toff) * _TS
    s2 = jnp.sum(jnp.where(lane < ns, minv, 0.0))

    @pl.when(first)
    def _():
        sd2_ref[0, 0, lev] = s2

    @pl.when(jnp.logical_not(first))
    def _():
        sd2_ref[0, 0, lev] = sd2_ref[0, 0, lev] + s2

    @pl.when(last)
    def _():
        sd1_ref[0, 0, lev] = jnp.sum(jnp.min(dmw_ref[...], axis=1))


def _chamfer_all(g24, pred_ts):
    npc = _TOT_T * _TS
    return pl.pallas_call(
        _chamfer_body,
        grid=(_B, _TOT_T),
        in_specs=[
            pl.BlockSpec((1, _NGT, 24), lambda b, t: (b, 0, 0)),
            pl.BlockSpec((1, 3, _TS), lambda b, t: (b, 0, 0)),
            pl.BlockSpec((1, 3, _TS),
                         lambda b, t: (b, 0, jnp.clip(t - 1, 0, 1))),
            pl.BlockSpec((1, 3, _TS),
                         lambda b, t: (b, 0, jnp.clip(t - 3, 0, 4))),
        ],
        out_specs=[
            pl.BlockSpec((1, 1, 3), lambda b, t: (b, 0, 0),
                         memory_space=pltpu.SMEM),
            pl.BlockSpec((1, 1, 3), lambda b, t: (b, 0, 0),
                         memory_space=pltpu.SMEM),
            pl.BlockSpec((1, 1, _TS), lambda b, t: (b, 0, t)),
        ],
        out_shape=[
            jax.ShapeDtypeStruct((_B, 1, 3), jnp.float32),
            jax.ShapeDtypeStruct((_B, 1, 3), jnp.float32),
            jax.ShapeDtypeStruct((_B, 1, npc), jnp.int32),
        ],
        scratch_shapes=[pltpu.VMEM((_NGT, 128), jnp.float32)],
    )(g24, pred_ts[0], pred_ts[1], pred_ts[2])


def _sc_body(p0, b0, p1, b1, p2, b2, nrm, i2_0, i2_1, i2_2,
             e00, e10, e01, e11, e02, e12,
             ln0, ln1, ln2, cn0, cn1, cn2,
             tri0, tri1, tri2, parts,
             px, py, pz, bx, by, bz, nx, ny, nz, i2v,
             e0v, e1v, lnv, cntv, av, cv, n2v, accv, sem):
    cid = lax.axis_index("c")
    sid = lax.axis_index("s")
    wid = sid * 2 + cid

    zero16 = jnp.zeros((16,), jnp.float32)
    for q in range(5):
        accv[pl.ds(q * 16, 16)] = zero16

    preds = (p0, p1, p2)
    befs = (b0, b1, b2)
    i2s = (i2_0, i2_1, i2_2)
    e0s = (e00, e01, e02)
    e1s = (e10, e11, e12)
    lns = (ln0, ln1, ln2)
    cns = (cn0, cn1, cn2)
    tris = (tri0, tri1, tri2)

    for b in range(_B):
        hn = [pltpu.async_copy(nrm.at[pl.ds((b * 3 + 0) * _NGT, _NGT)], nx, sem),
              pltpu.async_copy(nrm.at[pl.ds((b * 3 + 1) * _NGT, _NGT)], ny, sem),
              pltpu.async_copy(nrm.at[pl.ds((b * 3 + 2) * _NGT, _NGT)], nz, sem)]
        for h in hn:
            h.wait()
        for lev in range(3):
            nsp = _NSP[lev]
            ech = _ECH[lev]
            vch = _VCH[lev]
            ebase = wid * ech
            vbase = wid * vch
            pb = (b * 3) * nsp
            hs = [
                pltpu.async_copy(preds[lev].at[pl.ds(pb, nsp)],
                                 px.at[pl.ds(0, nsp)], sem),
                pltpu.async_copy(preds[lev].at[pl.ds(pb + nsp, nsp)],
                                 py.at[pl.ds(0, nsp)], sem),
                pltpu.async_copy(preds[lev].at[pl.ds(pb + 2 * nsp, nsp)],
                                 pz.at[pl.ds(0, nsp)], sem),
                pltpu.async_copy(befs[lev].at[pl.ds(pb, nsp)],
                                 bx.at[pl.ds(0, nsp)], sem),
                pltpu.async_copy(befs[lev].at[pl.ds(pb + nsp, nsp)],
                                 by.at[pl.ds(0, nsp)], sem),
                pltpu.async_copy(befs[lev].at[pl.ds(pb + 2 * nsp, nsp)],
                                 bz.at[pl.ds(0, nsp)], sem),
                pltpu.async_copy(i2s[lev].at[pl.ds(b * nsp, nsp)],
                                 i2v.at[pl.ds(0, nsp)], sem),
                pltpu.async_copy(e0s[lev].at[pl.ds(ebase, ech)],
                                 e0v.at[pl.ds(0, ech)], sem),
                pltpu.async_copy(e1s[lev].at[pl.ds(ebase, ech)],
                                 e1v.at[pl.ds(0, ech)], sem),
                pltpu.async_copy(cns[lev].at[pl.ds(vbase, vch)],
                                 cntv.at[pl.ds(0, vch)], sem),
            ]
            for j in range(8):
                hs.append(pltpu.async_copy(
                    lns[lev].at[pl.ds(j * nsp + vbase, vch)],
                    lnv.at[j, pl.ds(0, vch)], sem))
            for h in hs:
                h.wait()

            def edge_iter(i, carry):
                off = i * 16
                e0 = e0v[pl.ds(off, 16)]
                e1 = e1v[pl.ds(off, 16)]
                dx = plsc.load_gather(px, [e0]) - plsc.load_gather(px, [e1])
                dy = plsc.load_gather(py, [e0]) - plsc.load_gather(py, [e1])
                dz = plsc.load_gather(pz, [e0]) - plsc.load_gather(pz, [e1])
                ni = plsc.load_gather(i2v, [e0])
                gx = plsc.load_gather(nx, [ni])
                gy = plsc.load_gather(ny, [ni])
                gz = plsc.load_gather(nz, [ni])
                av[pl.ds(off, 16)] = dx * dx + dy * dy + dz * dz
                cv[pl.ds(off, 16)] = dx * gx + dy * gy + dz * gz
                n2v[pl.ds(off, 16)] = gx * gx + gy * gy + gz * gz
                return carry

            lax.fori_loop(0, ech // 16, edge_iter, 0)
            nep = _NEP[lev]
            tb = (b * 3) * nep + ebase
            ho = [
                pltpu.async_copy(av.at[pl.ds(0, ech)],
                                 tris[lev].at[pl.ds(tb, ech)], sem),
                pltpu.async_copy(cv.at[pl.ds(0, ech)],
                                 tris[lev].at[pl.ds(tb + nep, ech)], sem),
                pltpu.async_copy(n2v.at[pl.ds(0, ech)],
                                 tris[lev].at[pl.ds(tb + 2 * nep, ech)], sem),
            ]

            _LAPQ = lev
            _MOVQ = 2 + lev if lev > 0 else -1

            def lap_iter(i, carry):
                off = i * 16
                g = vbase + off
                dxv = bx[pl.ds(g, 16)] - px[pl.ds(g, 16)]
                dyv = by[pl.ds(g, 16)] - py[pl.ds(g, 16)]
                dzv = bz[pl.ds(g, 16)] - pz[pl.ds(g, 16)]
                sx = jnp.zeros((16,), jnp.float32)
                sy = jnp.zeros((16,), jnp.float32)
                sz = jnp.zeros((16,), jnp.float32)
                for j in range(8):
                    nb = lnv[j, pl.ds(off, 16)]
                    vf = jnp.where(nb >= 0, 1.0, 0.0).astype(jnp.float32)
                    nbs = jnp.maximum(nb, 0)
                    sx = sx + (plsc.load_gather(bx, [nbs])
                               - plsc.load_gather(px, [nbs])) * vf
                    sy = sy + (plsc.load_gather(by, [nbs])
                               - plsc.load_gather(py, [nbs])) * vf
                    sz = sz + (plsc.load_gather(bz, [nbs])
                               - plsc.load_gather(pz, [nbs])) * vf
                cc = cntv[pl.ds(off, 16)]
                lx = dxv - sx / cc
                ly = dyv - sy / cc
                lz = dzv - sz / cc
                lo = _LAPQ * 16
                accv[pl.ds(lo, 16)] = (accv[pl.ds(lo, 16)]
                                       + lx * lx + ly * ly + lz * lz)
                if _MOVQ >= 0:
                    mo = _MOVQ * 16
                    accv[pl.ds(mo, 16)] = (accv[pl.ds(mo, 16)]
                                           + dxv * dxv + dyv * dyv + dzv * dzv)
                return carry

            lax.fori_loop(0, vch // 16, lap_iter, 0)
            for h in ho:
                h.wait()

    pltpu.sync_copy(accv, parts.at[pl.ds(wid * 80, 80)])


def _sc_gather(pred_ts, bef_ts, nrm_t, idx2s, e0s, e1s, lapns, cnts):
    mesh = plsc.VectorSubcoreMesh(core_axis_name="c", subcore_axis_name="s",
                                  num_cores=2, num_subcores=16)
    out_type = [
        jax.ShapeDtypeStruct((_B * 3 * _NEP[0],), jnp.float32),
        jax.ShapeDtypeStruct((_B * 3 * _NEP[1],), jnp.float32),
        jax.ShapeDtypeStruct((_B * 3 * _NEP[2],), jnp.float32),
        jax.ShapeDtypeStruct((_NW * 80,), jnp.float32),
    ]
    scratch = [
        pltpu.VMEM((2560,), jnp.float32),
        pltpu.VMEM((2560,), jnp.float32),
        pltpu.VMEM((2560,), jnp.float32),
        pltpu.VMEM((2560,), jnp.float32),
        pltpu.VMEM((2560,), jnp.float32),
        pltpu.VMEM((2560,), jnp.float32),
        pltpu.VMEM((2048,), jnp.float32),
        pltpu.VMEM((2048,), jnp.float32),
        pltpu.VMEM((2048,), jnp.float32),
        pltpu.VMEM((2560,), jnp.int32),
        pltpu.VMEM((256,), jnp.int32),
        pltpu.VMEM((256,), jnp.int32),
        pltpu.VMEM((8, 128), jnp.int32),
        pltpu.VMEM((128,), jnp.float32),
        pltpu.VMEM((256,), jnp.float32),
        pltpu.VMEM((256,), jnp.float32),
        pltpu.VMEM((256,), jnp.float32),
        pltpu.VMEM((80,), jnp.float32),
        pltpu.SemaphoreType.DMA,
    ]
    f = pl.kernel(_sc_body, out_type=out_type, mesh=mesh,
                  scratch_types=scratch,
                  compiler_params=pltpu.CompilerParams(
                      use_tc_tiling_on_sc=False,
                      needs_layout_passes=False))
    return f(pred_ts[0], bef_ts[0], pred_ts[1], bef_ts[1],
             pred_ts[2], bef_ts[2], nrm_t,
             idx2s[0], idx2s[1], idx2s[2],
             e0s[0], e1s[0], e0s[1], e1s[1], e0s[2], e1s[2],
             lapns[0], lapns[1], lapns[2], cnts[0], cnts[1], cnts[2])


def _bce_body(gt_ref, p_ref, out_ref):
    p = jnp.clip(p_ref[...], 1e-7, 1.0 - 1e-7)
    gt = gt_ref[...]
    out_ref[0, 0, 0] = jnp.sum(gt * jnp.log(p)
                               + (1.0 - gt) * jnp.log(1.0 - p))


def _bce(gt_img, pred_img):
    return pl.pallas_call(
        _bce_body,
        grid=(3,),
        in_specs=[
            pl.BlockSpec((392, 512), lambda i: (i, 0)),
            pl.BlockSpec((392, 512), lambda i: (i, 0)),
        ],
        out_specs=pl.BlockSpec((1, 1, 1), lambda i: (i, 0, 0),
                               memory_space=pltpu.SMEM),
        out_shape=jax.ShapeDtypeStruct((3, 1, 1), jnp.float32),
        compiler_params=pltpu.CompilerParams(
            dimension_semantics=("parallel",)),
    )(gt_img, pred_img)


def _combine_body(sd1a, sd2a, tri0, tri1, tri2, parts, bs, *outs):
    sd1 = tuple(sum(sd1a[b, 0, i] for b in range(_B)) for i in range(3))
    sd2 = tuple(sum(sd2a[b, 0, i] for b in range(_B)) for i in range(3))
    tris = (tri0, tri1, tri2)
    chamfer = jnp.float32(0.0)
    edge = jnp.float32(0.0)
    normal = jnp.float32(0.0)
    lap = jnp.float32(0.0)
    move = jnp.float32(0.0)
    for i in range(3):
        ns = _NSL[i]
        ne = _NEL[i]
        a = tris[i][:, 0, :]
        c = tris[i][:, 1, :]
        n2 = tris[i][:, 2, :]
        cos = jnp.abs(c) / (jnp.maximum(jnp.sqrt(a), _EPS)
                            * jnp.maximum(jnp.sqrt(n2), _EPS))
        normal = normal + jnp.sum(cos) / (_B * ne)
        edge = edge + jnp.sum(a) / (_B * ne)
        chamfer = chamfer + _CHAMFER_W[i] * (
            sd1[i] / (_B * _NGT) + _CHAMFER_OPP_W * sd2[i] / (_B * ns))
        lap = lap + _LAP_CONST[i] * jnp.sum(parts[:, i, :]) / (_B * ns)
        if i > 0:
            move = move + _LAP_CONST[i] * jnp.sum(parts[:, 2 + i, :]) / (_B * ns)
    image = -(bs[0, 0, 0] + bs[1, 0, 0] + bs[2, 0, 0]) / (_B * 3 * 224 * 224)
    loss = (chamfer + image * _RECONST_W + _LAP_W * lap + _MOVE_W * move
            + _EDGE_W * edge + _NORMAL_W * normal)
    vals = (loss, image, chamfer, edge, lap, move, normal)
    for r, v in zip(outs, vals):
        r[0, 0] = v


def _combine(sd1a, sd2a, tris, parts, bs):
    smem = pl.BlockSpec(memory_space=pltpu.SMEM)
    in_specs = [smem, smem] + [pl.BlockSpec(t.shape, lambda: (0, 0, 0))
                               for t in tris]
    in_specs += [pl.BlockSpec(parts.shape, lambda: (0, 0, 0)), smem]
    return pl.pallas_call(
        _combine_body,
        in_specs=in_specs,
        out_specs=[smem] * 7,
        out_shape=[jax.ShapeDtypeStruct((1, 1), jnp.float32)] * 7,
    )(sd1a, sd2a, *tris, parts, bs)


def kernel(gt_points, gt_normals, gt_images, pred_reconst,
           pred_coord_0, pred_coord_1, pred_coord_2,
           pred_before_0, pred_before_1, pred_before_2,
           edges_0, edges_1, edges_2,
           lap_idx_0, lap_idx_1, lap_idx_2):
    gt_p = gt_points.astype(jnp.float32)
    nrm_t = jnp.transpose(gt_normals, (0, 2, 1)).astype(jnp.float32)
    preds = (pred_coord_0, pred_coord_1, pred_coord_2)
    befs = (pred_before_0, pred_before_1, pred_before_2)
    edges = (edges_0, edges_1, edges_2)
    laps = (lap_idx_0, lap_idx_1, lap_idx_2)

    pred_ts, bef_ts, e0s, e1s, lapns, cnts = [], [], [], [], [], []
    for i in range(3):
        nsp = _NSP[i]
        nep = _NEP[i]
        pred_t = _pad_last(jnp.transpose(preds[i], (0, 2, 1)), nsp, _BIG)
        bef_t = _pad_last(jnp.transpose(befs[i], (0, 2, 1)), nsp, _BIG)
        e = edges[i].astype(jnp.int32)
        e0 = _pad_last(e[:, 0], nep, 0)
        e1 = _pad_last(e[:, 1], nep, 0)
        li = laps[i].astype(jnp.int32)
        lapn = _pad_last(jnp.transpose(li[:, :8], (1, 0)), nsp, -1)
        cnt = _pad_last(li[:, 9].astype(jnp.float32), nsp, 1.0)
        pred_ts.append(pred_t)
        bef_ts.append(bef_t)
        e0s.append(e0)
        e1s.append(e1)
        lapns.append(lapn)
        cnts.append(cnt)

    gn = jnp.sum(gt_p * gt_p, axis=-1, keepdims=True)
    g5 = jnp.concatenate(
        [-2.0 * gt_p, gn, jnp.ones((_B, _NGT, 1), jnp.float32),
         jnp.zeros((_B, _NGT, 3), jnp.float32)], axis=-1)
    g5hi = g5.astype(jnp.bfloat16)
    g5lo = (g5 - g5hi.astype(jnp.float32)).astype(jnp.bfloat16)
    g24 = jnp.concatenate([g5hi, g5hi, g5lo], axis=-1)
    sd1a, sd2a, idx2c = _chamfer_all(g24, pred_ts)
    idx2s = [idx2c[:, :, _TOFF[i] * _TS:_TOFF[i] * _TS + _NSP[i]]
             for i in range(3)]

    pred_fs = [jnp.reshape(p, (-1,)) for p in pred_ts]
    bef_fs = [jnp.reshape(p, (-1,)) for p in bef_ts]
    nrm_f = jnp.reshape(nrm_t, (-1,))
    idx2_fs = [jnp.reshape(ix, (-1,)) for ix in idx2s]
    lapn_fs = [jnp.reshape(ln, (-1,)) for ln in lapns]
    tri0, tri1, tri2, parts = _sc_gather(pred_fs, bef_fs, nrm_f, idx2_fs,
                                         e0s, e1s, lapn_fs, cnts)
    tri0 = jnp.reshape(tri0, (_B, 3, _NEP[0]))
    tri1 = jnp.reshape(tri1, (_B, 3, _NEP[1]))
    tri2 = jnp.reshape(tri2, (_B, 3, _NEP[2]))
    parts = jnp.reshape(parts, (_NW, 5, 16))

    gt_img = jnp.reshape(gt_images, (1176, 512))
    p_img = jnp.reshape(pred_reconst, (1176, 512))
    bs = _bce(gt_img, p_img)

    outs = _combine(sd1a, sd2a, (tri0, tri1, tri2), parts, bs)
    return tuple(jnp.reshape(o, ()) for o in outs)
